# baseline (speedup 1.0000x reference)
import functools

import jax
import jax.numpy as jnp
from jax import lax
from jax.experimental import pallas as pl
from jax.experimental.pallas import tpu as pltpu
from jax.experimental.pallas import tpu_sc as plsc

NC = 2
NS = 16
NW = NC * NS
BD = 128
BG = 120
D = 128
RB = 1024
NB = 8
NBG = 3


def _sc_mesh():
    return plsc.VectorSubcoreMesh(
        core_axis_name="c", subcore_axis_name="s", num_cores=NC, num_subcores=NS
    )


def _deg_body(P, NP, nb, earr_hbm, out_hbm, acc_sh, ebuf, ones_v, zero_v, *sems):
    ss = list(sems[:2 * nb])
    se = sems[2 * nb]
    c = lax.axis_index("c")
    s = lax.axis_index("s")
    wid = c * NS + s
    rpt = NP // NS
    EH = 8

    def _fill(i, _):
        ones_v[pl.ds(i * 16, 16)] = jnp.ones((16,), jnp.float32)
        zero_v[pl.ds(i * 16, 16)] = jnp.zeros((16,), jnp.float32)
        return _

    lax.fori_loop(0, BD // 16, _fill, None)

    def _zero(i, _):
        pltpu.sync_copy(zero_v, acc_sh.at[pl.ds(s * rpt + i * BD, BD)])
        return _

    lax.fori_loop(0, rpt // BD, _zero, None)
    plsc.subcore_barrier()

    pltpu.sync_copy(earr_hbm.at[pl.ds(EH * wid * P, EH)], ebuf.at[pl.ds(0, EH)])

    def _blk(pp, _):
        for pq in range(2):
            p = 2 * pp + pq
            gb = (p % 3) * EH
            gbn = ((p + 1) % 3) * EH
            for k in range(nb):
                if k == 0:

                    @pl.when(p > 0)
                    def _w():
                        pltpu.make_async_copy(
                            earr_hbm.at[pl.ds(EH * wid * P, EH)],
                            ebuf.at[pl.ds(gb, EH)], se).wait()

                @pl.when(p > 1)
                def _wprev():
                    pltpu.make_async_copy(
                        ones_v.at[pl.ds(0, BG)],
                        acc_sh.at[ebuf.at[gbn + 2 * k + 1]],
                        ss[pq * nb + k]).wait()

                pltpu.async_copy(
                    ones_v.at[pl.ds(0, BG)],
                    acc_sh.at[ebuf.at[gb + 2 * k + 1]], ss[pq * nb + k],
                    add=True)
                if k == nb - 1:

                    @pl.when(p + 1 < P)
                    def _ld():
                        pltpu.async_copy(
                            earr_hbm.at[pl.ds(EH * (wid * P + p + 1), EH)],
                            ebuf.at[pl.ds(gbn, EH)], se)

        return _

    lax.fori_loop(0, P // 2, _blk, None)
    for q in (P - 2, P - 1):
        for k in range(nb):
            pltpu.make_async_copy(
                ones_v.at[pl.ds(0, BG)],
                acc_sh.at[ebuf.at[(q % 3) * EH + 2 * k + 1]],
                ss[(q % 2) * nb + k]).wait()
    plsc.subcore_barrier()
    pltpu.sync_copy(acc_sh.at[pl.ds(s * rpt, rpt)], out_hbm.at[c, pl.ds(s * rpt, rpt)])


def _sc_degree(earr, NP, P):
    nb = NBG
    kfn = pl.kernel(
        functools.partial(_deg_body, P, NP, nb),
        out_type=jax.ShapeDtypeStruct((NC, NP), jnp.float32),
        mesh=_sc_mesh(),
        scratch_types=[
            pltpu.VMEM_SHARED((NP,), jnp.float32),
            pltpu.VMEM((24, BG), jnp.int32),
            pltpu.VMEM((BD,), jnp.float32),
            pltpu.VMEM((BD,), jnp.float32),
        ] + [pltpu.SemaphoreType.DMA] * (2 * nb + 1),
    )
    return kfn(earr)


def _gcn_body(T, NP, nb, hp_hbm, earr_hbm, out_hbm, acc_sh, ebuf, *bufs):
    rows = list(bufs[:nb])
    sg = list(bufs[nb:2 * nb])
    ss = list(bufs[2 * nb:3 * nb])
    se = bufs[3 * nb]
    c = lax.axis_index("c")
    s = lax.axis_index("s")
    wid = c * NS + s
    rpt = NP // NS
    qbase = wid * T
    P = T // nb

    pltpu.sync_copy(hp_hbm.at[pl.ds(s * rpt, rpt)], acc_sh.at[pl.ds(s * rpt, rpt)])
    plsc.subcore_barrier()

    EH = 8
    pltpu.sync_copy(earr_hbm.at[pl.ds(EH * wid * P, EH)], ebuf.at[pl.ds(0, EH)])
    for k in range(nb - 1):
        pltpu.async_copy(hp_hbm.at[ebuf.at[2 * k]], rows[k], sg[k])

    def _blk(p, _):
        gb = (p % 2) * EH
        gbn = ((p + 1) % 2) * EH
        for k in range(nb):
            bprev = (k + nb - 1) % nb
            pltpu.make_async_copy(
                hp_hbm.at[ebuf.at[gb + 2 * k]], rows[k], sg[k]).wait()
            pltpu.async_copy(
                rows[k], acc_sh.at[ebuf.at[gb + 2 * k + 1]], ss[k], add=True)
            if k == 0:

                @pl.when(p > 0)
                def _wsc():
                    pltpu.make_async_copy(
                        rows[bprev],
                        acc_sh.at[ebuf.at[gbn + 2 * bprev + 1]], ss[bprev]).wait()

                @pl.when(p + 1 < P)
                def _ld():
                    pltpu.async_copy(
                        earr_hbm.at[pl.ds(EH * (wid * P + p + 1), EH)],
                        ebuf.at[pl.ds(gbn, EH)], se)

            else:
                pltpu.make_async_copy(
                    rows[bprev],
                    acc_sh.at[ebuf.at[gb + 2 * bprev + 1]], ss[bprev]).wait()
            if k == 0:
                pltpu.async_copy(
                    hp_hbm.at[ebuf.at[gb + 2 * (nb - 1)]], rows[nb - 1], sg[nb - 1])
            else:
                if k == 1:

                    @pl.when(p + 1 < P)
                    def _we():
                        pltpu.make_async_copy(
                            earr_hbm.at[pl.ds(EH * wid * P, EH)],
                            ebuf.at[pl.ds(gbn, EH)], se).wait()

                ku = k - 1

                @pl.when(p + 1 < P)
                def _g():
                    pltpu.async_copy(
                        hp_hbm.at[ebuf.at[gbn + 2 * ku]], rows[ku], sg[ku])

        return _

    lax.fori_loop(0, P, _blk, None)
    gb_last = ((P - 1) % 2) * EH
    pltpu.make_async_copy(
        rows[nb - 1],
        acc_sh.at[ebuf.at[gb_last + 2 * (nb - 1) + 1]], ss[nb - 1]).wait()
    plsc.subcore_barrier()
    pltpu.sync_copy(
        acc_sh.at[pl.ds(s * rpt, rpt)], out_hbm.at[c, pl.ds(s * rpt, rpt)]
    )


def _sc_gcn(hp, earr, NP, T):
    nb = NBG
    kfn = pl.kernel(
        functools.partial(_gcn_body, T, NP, nb),
        out_type=jax.ShapeDtypeStruct((NC, NP, D), jnp.float32),
        mesh=_sc_mesh(),
        scratch_types=(
            [pltpu.VMEM_SHARED((NP, D), jnp.float32),
             pltpu.VMEM((16, BG), jnp.int32)]
            + [pltpu.VMEM((BG, D), jnp.float32)] * nb
            + [pltpu.SemaphoreType.DMA] * (2 * nb + 1)
        ),
    )
    return kfn(hp, earr)


def _tc1_body(x_ref, w_ref, degt_ref, hp_ref, dinv_ref):
    d = degt_ref[:, 0:1] + degt_ref[:, 1:2] + 1.0
    dinv = lax.rsqrt(d)
    h = jnp.dot(x_ref[...], w_ref[...], preferred_element_type=jnp.float32)
    hp_ref[...] = h * dinv
    dinv_ref[...] = dinv


def _tc1(x_pad, W3, degT, NP):
    grid = NP // RB
    return pl.pallas_call(
        _tc1_body,
        grid=(grid,),
        in_specs=[
            pl.BlockSpec((RB, D), lambda i: (i, 0)),
            pl.BlockSpec((D, D), lambda i: (0, 0)),
            pl.BlockSpec((RB, NC), lambda i: (i, 0)),
        ],
        out_specs=[
            pl.BlockSpec((RB, D), lambda i: (i, 0)),
            pl.BlockSpec((RB, 1), lambda i: (i, 0)),
        ],
        out_shape=[
            jax.ShapeDtypeStruct((NP, D), jnp.float32),
            jax.ShapeDtypeStruct((NP, 1), jnp.float32),
        ],
    )(x_pad, W3, degT)


def _ln_relu(pre, lnw, lnb):
    mu = jnp.mean(pre, axis=1, keepdims=True)
    var = jnp.mean((pre - mu) ** 2, axis=1, keepdims=True)
    y = (pre - mu) * lax.rsqrt(var + 1e-5) * lnw + lnb
    return jnp.maximum(y, 0.0)


def _tc2_body(acc_ref, hp_ref, dinv_ref, b_ref, lnw_ref, lnb_ref, w_ref,
              z_ref, hpn_ref):
    a = acc_ref[0, :, :] + acc_ref[1, :, :] - hp_ref[...]
    pre = a * dinv_ref[...] + b_ref[...]
    z = _ln_relu(pre, lnw_ref[...], lnb_ref[...])
    z_ref[...] = z
    hpn_ref[...] = (
        jnp.dot(z, w_ref[...], preferred_element_type=jnp.float32) * dinv_ref[...]
    )


def _tc2(acc, hp, dinv, b, lnw, lnb, W, NP, N):
    grid = NP // RB
    return pl.pallas_call(
        _tc2_body,
        grid=(grid,),
        in_specs=[
            pl.BlockSpec((NC, RB, D), lambda i: (0, i, 0)),
            pl.BlockSpec((RB, D), lambda i: (i, 0)),
            pl.BlockSpec((RB, 1), lambda i: (i, 0)),
            pl.BlockSpec((1, D), lambda i: (0, 0)),
            pl.BlockSpec((1, D), lambda i: (0, 0)),
            pl.BlockSpec((1, D), lambda i: (0, 0)),
            pl.BlockSpec((D, D), lambda i: (0, 0)),
        ],
        out_specs=[
            pl.BlockSpec((RB, D), lambda i: (i, 0)),
            pl.BlockSpec((RB, D), lambda i: (i, 0)),
        ],
        out_shape=[
            jax.ShapeDtypeStruct((N, D), jnp.float32),
            jax.ShapeDtypeStruct((NP, D), jnp.float32),
        ],
    )(acc, hp, dinv, b, lnw, lnb, W)


def _tc3_body(acc_ref, hp_ref, dinv_ref, b_ref, lnw_ref, lnb_ref, z_ref):
    a = acc_ref[0, :, :] + acc_ref[1, :, :] - hp_ref[...]
    pre = a * dinv_ref[...] + b_ref[...]
    z_ref[...] = _ln_relu(pre, lnw_ref[...], lnb_ref[...])


def _tc3(acc, hp, dinv, b, lnw, lnb, NP, N):
    grid = NP // RB
    return pl.pallas_call(
        _tc3_body,
        grid=(grid,),
        in_specs=[
            pl.BlockSpec((NC, RB, D), lambda i: (0, i, 0)),
            pl.BlockSpec((RB, D), lambda i: (i, 0)),
            pl.BlockSpec((RB, 1), lambda i: (i, 0)),
            pl.BlockSpec((1, D), lambda i: (0, 0)),
            pl.BlockSpec((1, D), lambda i: (0, 0)),
            pl.BlockSpec((1, D), lambda i: (0, 0)),
        ],
        out_specs=pl.BlockSpec((RB, D), lambda i: (i, 0)),
        out_shape=jax.ShapeDtypeStruct((N, D), jnp.float32),
    )(acc, hp, dinv, b, lnw, lnb)


def kernel(x, all_edges, W3, b3, W4, b4, ln_w, ln_b):
    N = x.shape[0]
    E = all_edges.shape[1]

    NP = ((N + RB - 1) // RB) * RB
    assert NP % (NS * BD) == 0 and NP > N

    src = all_edges[0].astype(jnp.int32)
    dst = all_edges[1].astype(jnp.int32)

    T = (E + NW * BG * NBG - 1) // (NW * BG * NBG) * NBG
    pad = T * NW * BG - E
    fill = jnp.arange(pad, dtype=jnp.int32)
    src_pad = jnp.concatenate([src, fill % N])
    dst_pad = jnp.concatenate([dst, N + fill % (NP - N)])
    P = T // NBG
    inter = jnp.stack(
        [src_pad.reshape(NW, P, NBG, BG), dst_pad.reshape(NW, P, NBG, BG)],
        axis=3,
    ).reshape(NW, P, 2 * NBG, BG)
    earr = jnp.pad(
        inter, ((0, 0), (0, 0), (0, 8 - 2 * NBG), (0, 0))
    ).reshape(NW * P * 8, BG)

    b3r = b3.reshape(1, D)
    b4r = b4.reshape(1, D)
    lnwr = ln_w.reshape(1, D)
    lnbr = ln_b.reshape(1, D)

    deg = _sc_degree(earr, NP, P)
    degT = deg.T

    h3p, dinv = _tc1(x, W3, degT, NP)
    acc1 = _sc_gcn(h3p, earr, NP, T)
    z1, h4p = _tc2(acc1, h3p, dinv, b3r, lnwr, lnbr, W4, NP, N)
    acc2 = _sc_gcn(h4p, earr, NP, T)
    z2 = _tc3(acc2, h4p, dinv, b4r, lnwr, lnbr, NP, N)

    return (z1, z2)


# speedup vs baseline: 1.0608x; 1.0608x over previous
import functools

import jax
import jax.numpy as jnp
from jax import lax
from jax.experimental import pallas as pl
from jax.experimental.pallas import tpu as pltpu
from jax.experimental.pallas import tpu_sc as plsc

NC = 2
NS = 16
NW = NC * NS
BD = 128
BG = 120
D = 128
RB = 1024
NB = 8
NBG = 3


def _sc_mesh():
    return plsc.VectorSubcoreMesh(
        core_axis_name="c", subcore_axis_name="s", num_cores=NC, num_subcores=NS
    )


def _deg_body(T, NP, darr_hbm, out_hbm, acc_sh, ebuf, ones_v, zero_v, *sems):
    ss = list(sems[:NB])
    se = sems[NB]
    c = lax.axis_index("c")
    s = lax.axis_index("s")
    wid = c * NS + s
    rpt = NP // NS
    qbase = wid * T
    P = T // NB

    def _fill(i, _):
        ones_v[pl.ds(i * 16, 16)] = jnp.ones((16,), jnp.float32)
        zero_v[pl.ds(i * 16, 16)] = jnp.zeros((16,), jnp.float32)
        return _

    lax.fori_loop(0, BD // 16, _fill, None)

    def _zero(i, _):
        pltpu.sync_copy(zero_v, acc_sh.at[pl.ds(s * rpt + i * BD, BD)])
        return _

    lax.fori_loop(0, rpt // BD, _zero, None)
    plsc.subcore_barrier()

    pltpu.sync_copy(darr_hbm.at[pl.ds(qbase, NB)], ebuf.at[pl.ds(0, NB)])

    def _blk(p, _):
        gb = (p % 2) * NB
        gbn = ((p + 1) % 2) * NB
        for k in range(NB):
            if k == 0:

                @pl.when(p > 0)
                def _w():
                    pltpu.make_async_copy(
                        darr_hbm.at[pl.ds(qbase, NB)],
                        ebuf.at[pl.ds(gb, NB)], se).wait()

            @pl.when(p > 0)
            def _wprev():
                pltpu.make_async_copy(
                    ones_v, acc_sh.at[ebuf.at[gbn + k]], ss[k]).wait()

            pltpu.async_copy(
                ones_v, acc_sh.at[ebuf.at[gb + k]], ss[k], add=True)
            if k == NB - 1:

                @pl.when(p + 1 < P)
                def _ld():
                    pltpu.async_copy(
                        darr_hbm.at[pl.ds(qbase + NB * (p + 1), NB)],
                        ebuf.at[pl.ds(gbn, NB)], se)

        return _

    lax.fori_loop(0, P, _blk, None)
    gb_last = ((P - 1) % 2) * NB
    for k in range(NB):
        pltpu.make_async_copy(
            ones_v, acc_sh.at[ebuf.at[gb_last + k]], ss[k]).wait()
    plsc.subcore_barrier()
    pltpu.sync_copy(acc_sh.at[pl.ds(s * rpt, rpt)], out_hbm.at[c, pl.ds(s * rpt, rpt)])


def _sc_degree(darr, NP, T):
    kfn = pl.kernel(
        functools.partial(_deg_body, T, NP),
        out_type=jax.ShapeDtypeStruct((NC, NP), jnp.float32),
        mesh=_sc_mesh(),
        scratch_types=[
            pltpu.VMEM_SHARED((NP,), jnp.float32),
            pltpu.VMEM((2 * NB, BD), jnp.int32),
            pltpu.VMEM((BD,), jnp.float32),
            pltpu.VMEM((BD,), jnp.float32),
        ] + [pltpu.SemaphoreType.DMA] * (NB + 1),
    )
    return kfn(darr)


def _gcn_body(T, NP, nb, hp_hbm, earr_hbm, out_hbm, acc_sh, ebuf, *bufs):
    rows = list(bufs[:nb])
    sg = list(bufs[nb:2 * nb])
    ss = list(bufs[2 * nb:3 * nb])
    se = bufs[3 * nb]
    c = lax.axis_index("c")
    s = lax.axis_index("s")
    wid = c * NS + s
    rpt = NP // NS
    qbase = wid * T
    P = T // nb

    pltpu.sync_copy(hp_hbm.at[pl.ds(s * rpt, rpt)], acc_sh.at[pl.ds(s * rpt, rpt)])
    plsc.subcore_barrier()

    EH = 8
    pltpu.sync_copy(earr_hbm.at[pl.ds(EH * wid * P, EH)], ebuf.at[pl.ds(0, EH)])
    for k in range(nb - 1):
        pltpu.async_copy(hp_hbm.at[ebuf.at[2 * k]], rows[k], sg[k])

    def _blk(p, _):
        gb = (p % 2) * EH
        gbn = ((p + 1) % 2) * EH
        for k in range(nb):
            bprev = (k + nb - 1) % nb
            pltpu.make_async_copy(
                hp_hbm.at[ebuf.at[gb + 2 * k]], rows[k], sg[k]).wait()
            pltpu.async_copy(
                rows[k], acc_sh.at[ebuf.at[gb + 2 * k + 1]], ss[k], add=True)
            if k == 0:

                @pl.when(p > 0)
                def _wsc():
                    pltpu.make_async_copy(
                        rows[bprev],
                        acc_sh.at[ebuf.at[gbn + 2 * bprev + 1]], ss[bprev]).wait()

                @pl.when(p + 1 < P)
                def _ld():
                    pltpu.async_copy(
                        earr_hbm.at[pl.ds(EH * (wid * P + p + 1), EH)],
                        ebuf.at[pl.ds(gbn, EH)], se)

            else:
                pltpu.make_async_copy(
                    rows[bprev],
                    acc_sh.at[ebuf.at[gb + 2 * bprev + 1]], ss[bprev]).wait()
            if k == 0:
                pltpu.async_copy(
                    hp_hbm.at[ebuf.at[gb + 2 * (nb - 1)]], rows[nb - 1], sg[nb - 1])
            else:
                if k == 1:

                    @pl.when(p + 1 < P)
                    def _we():
                        pltpu.make_async_copy(
                            earr_hbm.at[pl.ds(EH * wid * P, EH)],
                            ebuf.at[pl.ds(gbn, EH)], se).wait()

                ku = k - 1

                @pl.when(p + 1 < P)
                def _g():
                    pltpu.async_copy(
                        hp_hbm.at[ebuf.at[gbn + 2 * ku]], rows[ku], sg[ku])

        return _

    lax.fori_loop(0, P, _blk, None)
    gb_last = ((P - 1) % 2) * EH
    pltpu.make_async_copy(
        rows[nb - 1],
        acc_sh.at[ebuf.at[gb_last + 2 * (nb - 1) + 1]], ss[nb - 1]).wait()
    plsc.subcore_barrier()
    pltpu.sync_copy(
        acc_sh.at[pl.ds(s * rpt, rpt)], out_hbm.at[c, pl.ds(s * rpt, rpt)]
    )


def _sc_gcn(hp, earr, NP, T):
    nb = NBG
    kfn = pl.kernel(
        functools.partial(_gcn_body, T, NP, nb),
        out_type=jax.ShapeDtypeStruct((NC, NP, D), jnp.float32),
        mesh=_sc_mesh(),
        scratch_types=(
            [pltpu.VMEM_SHARED((NP, D), jnp.float32),
             pltpu.VMEM((16, BG), jnp.int32)]
            + [pltpu.VMEM((BG, D), jnp.float32)] * nb
            + [pltpu.SemaphoreType.DMA] * (2 * nb + 1)
        ),
    )
    return kfn(hp, earr)


def _tc1_body(x_ref, w_ref, degt_ref, hp_ref, dinv_ref):
    d = degt_ref[:, 0:1] + degt_ref[:, 1:2] + 1.0
    dinv = lax.rsqrt(d)
    h = jnp.dot(x_ref[...], w_ref[...], preferred_element_type=jnp.float32)
    hp_ref[...] = h * dinv
    dinv_ref[...] = dinv


def _tc1(x_pad, W3, degT, NP):
    grid = NP // RB
    return pl.pallas_call(
        _tc1_body,
        grid=(grid,),
        in_specs=[
            pl.BlockSpec((RB, D), lambda i: (i, 0)),
            pl.BlockSpec((D, D), lambda i: (0, 0)),
            pl.BlockSpec((RB, NC), lambda i: (i, 0)),
        ],
        out_specs=[
            pl.BlockSpec((RB, D), lambda i: (i, 0)),
            pl.BlockSpec((RB, 1), lambda i: (i, 0)),
        ],
        out_shape=[
            jax.ShapeDtypeStruct((NP, D), jnp.float32),
            jax.ShapeDtypeStruct((NP, 1), jnp.float32),
        ],
    )(x_pad, W3, degT)


def _ln_relu(pre, lnw, lnb):
    mu = jnp.mean(pre, axis=1, keepdims=True)
    var = jnp.mean((pre - mu) ** 2, axis=1, keepdims=True)
    y = (pre - mu) * lax.rsqrt(var + 1e-5) * lnw + lnb
    return jnp.maximum(y, 0.0)


def _tc2_body(acc_ref, hp_ref, dinv_ref, b_ref, lnw_ref, lnb_ref, w_ref,
              z_ref, hpn_ref):
    a = acc_ref[0, :, :] + acc_ref[1, :, :] - hp_ref[...]
    pre = a * dinv_ref[...] + b_ref[...]
    z = _ln_relu(pre, lnw_ref[...], lnb_ref[...])
    z_ref[...] = z
    hpn_ref[...] = (
        jnp.dot(z, w_ref[...], preferred_element_type=jnp.float32) * dinv_ref[...]
    )


def _tc2(acc, hp, dinv, b, lnw, lnb, W, NP, N):
    grid = NP // RB
    return pl.pallas_call(
        _tc2_body,
        grid=(grid,),
        in_specs=[
            pl.BlockSpec((NC, RB, D), lambda i: (0, i, 0)),
            pl.BlockSpec((RB, D), lambda i: (i, 0)),
            pl.BlockSpec((RB, 1), lambda i: (i, 0)),
            pl.BlockSpec((1, D), lambda i: (0, 0)),
            pl.BlockSpec((1, D), lambda i: (0, 0)),
            pl.BlockSpec((1, D), lambda i: (0, 0)),
            pl.BlockSpec((D, D), lambda i: (0, 0)),
        ],
        out_specs=[
            pl.BlockSpec((RB, D), lambda i: (i, 0)),
            pl.BlockSpec((RB, D), lambda i: (i, 0)),
        ],
        out_shape=[
            jax.ShapeDtypeStruct((N, D), jnp.float32),
            jax.ShapeDtypeStruct((NP, D), jnp.float32),
        ],
    )(acc, hp, dinv, b, lnw, lnb, W)


def _tc3_body(acc_ref, hp_ref, dinv_ref, b_ref, lnw_ref, lnb_ref, z_ref):
    a = acc_ref[0, :, :] + acc_ref[1, :, :] - hp_ref[...]
    pre = a * dinv_ref[...] + b_ref[...]
    z_ref[...] = _ln_relu(pre, lnw_ref[...], lnb_ref[...])


def _tc3(acc, hp, dinv, b, lnw, lnb, NP, N):
    grid = NP // RB
    return pl.pallas_call(
        _tc3_body,
        grid=(grid,),
        in_specs=[
            pl.BlockSpec((NC, RB, D), lambda i: (0, i, 0)),
            pl.BlockSpec((RB, D), lambda i: (i, 0)),
            pl.BlockSpec((RB, 1), lambda i: (i, 0)),
            pl.BlockSpec((1, D), lambda i: (0, 0)),
            pl.BlockSpec((1, D), lambda i: (0, 0)),
            pl.BlockSpec((1, D), lambda i: (0, 0)),
        ],
        out_specs=pl.BlockSpec((RB, D), lambda i: (i, 0)),
        out_shape=jax.ShapeDtypeStruct((N, D), jnp.float32),
    )(acc, hp, dinv, b, lnw, lnb)


def kernel(x, all_edges, W3, b3, W4, b4, ln_w, ln_b):
    N = x.shape[0]
    E = all_edges.shape[1]

    NP = ((N + RB - 1) // RB) * RB
    assert NP % (NS * BD) == 0 and NP > N

    src = all_edges[0].astype(jnp.int32)
    dst = all_edges[1].astype(jnp.int32)

    Td = (E + NW * BD * NB - 1) // (NW * BD * NB) * NB
    padd = Td * NW * BD - E
    filld = jnp.arange(padd, dtype=jnp.int32)
    darr = jnp.concatenate([dst, N + filld % (NP - N)]).reshape(NW * Td, BD)

    T = (E + NW * BG * NBG - 1) // (NW * BG * NBG) * NBG
    pad = T * NW * BG - E
    fill = jnp.arange(pad, dtype=jnp.int32)
    src_pad = jnp.concatenate([src, fill % N])
    dst_pad = jnp.concatenate([dst, N + fill % (NP - N)])
    P = T // NBG
    inter = jnp.stack(
        [src_pad.reshape(NW, P, NBG, BG), dst_pad.reshape(NW, P, NBG, BG)],
        axis=3,
    ).reshape(NW, P, 2 * NBG, BG)
    earr = jnp.pad(
        inter, ((0, 0), (0, 0), (0, 8 - 2 * NBG), (0, 0))
    ).reshape(NW * P * 8, BG)

    b3r = b3.reshape(1, D)
    b4r = b4.reshape(1, D)
    lnwr = ln_w.reshape(1, D)
    lnbr = ln_b.reshape(1, D)

    deg = _sc_degree(darr, NP, Td)
    degT = deg.T

    h3p, dinv = _tc1(x, W3, degT, NP)
    acc1 = _sc_gcn(h3p, earr, NP, T)
    z1, h4p = _tc2(acc1, h3p, dinv, b3r, lnwr, lnbr, W4, NP, N)
    acc2 = _sc_gcn(h4p, earr, NP, T)
    z2 = _tc3(acc2, h4p, dinv, b4r, lnwr, lnbr, NP, N)

    return (z1, z2)
